# baseline (speedup 1.0000x reference)
import functools

import jax
import jax.numpy as jnp
from jax import lax
from jax.experimental import pallas as pl
from jax.experimental.pallas import tpu as pltpu
from jax.experimental.pallas import tpu_sc as plsc

N_NODES = 100000
N_EDGES = 6400000
OUT_F = 128
N_GRAPHS = 64

LANES = 128
K_ROWS = 8
CHUNKS = N_EDGES // (LANES * K_ROWS)
NW = 32
CHUNKS_PER_W = CHUNKS // NW
CHUNKS_EXTRA = CHUNKS % NW

TILE_N = 512
N_PAD = 100352
N_TILES = N_PAD // TILE_N
STRIPE = N_PAD // 16


def _sc_edge_body(x_hbm, ei_hbm, aggr_out, deg_out,
                  x_sp, aggr_sp, deg_sp,
                  idx_v, vals, ones_v, zbuf,
                  gsem, gsem2, ssem, lsem):
    c = lax.axis_index("c")
    s = lax.axis_index("s")
    wid = c * 16 + s

    def _zb(i, _):
        zbuf[pl.ds(i * 16, 16)] = jnp.zeros((16,), jnp.float32)
        return 0
    lax.fori_loop(0, STRIPE // 16, _zb, 0)
    for j in range(K_ROWS):
        for i in range(LANES // 16):
            ones_v[j, pl.ds(i * 16, 16)] = jnp.ones((16,), jnp.float32)

    st = s * STRIPE
    pltpu.async_copy(x_hbm.at[pl.ds(st, STRIPE)], x_sp.at[pl.ds(st, STRIPE)],
                     lsem).wait()
    pltpu.async_copy(zbuf, aggr_sp.at[pl.ds(st, STRIPE)], lsem).wait()
    pltpu.async_copy(zbuf, deg_sp.at[pl.ds(st, STRIPE)], lsem).wait()
    plsc.subcore_barrier()

    base = wid * CHUNKS_PER_W + jnp.minimum(wid, CHUNKS_EXTRA)
    nchunks = CHUNKS_PER_W + jnp.where(wid < CHUNKS_EXTRA, 1, 0)

    def drain(sem, target, nrows):
        for j in range(nrows):
            pltpu.make_async_copy(x_hbm.at[pl.ds(0, LANES)], target.at[j],
                                  sem).wait()

    pltpu.async_copy(ei_hbm.at[0, base], idx_v.at[pl.ds(0, K_ROWS)], lsem)
    pltpu.async_copy(ei_hbm.at[1, base], idx_v.at[pl.ds(K_ROWS, K_ROWS)], lsem)

    def round_body(r, _):
        p = lax.rem(r, 3)
        pm1 = lax.rem(r + 2, 3)
        src_r = p * 2 * K_ROWS
        dst_r = p * 2 * K_ROWS + K_ROWS
        dstm1_r = pm1 * 2 * K_ROWS + K_ROWS
        valm1_r = pm1 * K_ROWS

        @pl.when(jnp.logical_or(r == 1, r == nchunks + 1))
        def _():
            drain(ssem, vals, K_ROWS)

        @pl.when(jnp.logical_and(r >= 2, r <= nchunks))
        def _():
            drain(ssem, vals, 2 * K_ROWS)

        @pl.when(r < nchunks)
        def _():
            pltpu.make_async_copy(ei_hbm.at[0, base], idx_v.at[pl.ds(0, K_ROWS)],
                                  lsem).wait()
            pltpu.make_async_copy(ei_hbm.at[0, base], idx_v.at[pl.ds(0, K_ROWS)],
                                  lsem).wait()

            @pl.when(r + 1 < nchunks)
            def _():
                ck = base + r + 1
                pn_r = lax.rem(r + 1, 3) * 2 * K_ROWS
                pltpu.async_copy(ei_hbm.at[0, ck],
                                 idx_v.at[pl.ds(pn_r, K_ROWS)], lsem)
                pltpu.async_copy(ei_hbm.at[1, ck],
                                 idx_v.at[pl.ds(pn_r + K_ROWS, K_ROWS)], lsem)

            for j in range(K_ROWS):
                pltpu.async_copy(ones_v.at[j], deg_sp.at[idx_v.at[dst_r + j]],
                                 ssem, add=True)

            @pl.when(lax.rem(r, 2) == 0)
            def _():
                for j in range(K_ROWS):
                    pltpu.async_copy(x_sp.at[idx_v.at[src_r + j]],
                                     vals.at[p * K_ROWS + j], gsem)

            @pl.when(lax.rem(r, 2) == 1)
            def _():
                for j in range(K_ROWS):
                    pltpu.async_copy(x_sp.at[idx_v.at[src_r + j]],
                                     vals.at[p * K_ROWS + j], gsem2)

        @pl.when(jnp.logical_and(r >= 1, r <= nchunks))
        def _():
            @pl.when(lax.rem(r, 2) == 1)
            def _():
                drain(gsem, vals, K_ROWS)

            @pl.when(lax.rem(r, 2) == 0)
            def _():
                drain(gsem2, vals, K_ROWS)

            for j in range(K_ROWS):
                pltpu.async_copy(vals.at[valm1_r + j],
                                 aggr_sp.at[idx_v.at[dstm1_r + j]],
                                 ssem, add=True)
        return 0
    lax.fori_loop(0, nchunks + 2, round_body, 0)

    plsc.subcore_barrier()

    pltpu.async_copy(aggr_sp.at[pl.ds(st, STRIPE)],
                     aggr_out.at[pl.ds(c * N_PAD + st, STRIPE)], lsem).wait()
    pltpu.async_copy(deg_sp.at[pl.ds(st, STRIPE)],
                     deg_out.at[pl.ds(c * N_PAD + st, STRIPE)], lsem).wait()


_sc_edge_kernel = functools.partial(
    pl.kernel,
    out_type=[jax.ShapeDtypeStruct((2 * N_PAD,), jnp.float32),
              jax.ShapeDtypeStruct((2 * N_PAD,), jnp.float32)],
    mesh=plsc.VectorSubcoreMesh(core_axis_name="c", subcore_axis_name="s"),
    scratch_types=[
        pltpu.VMEM_SHARED((N_PAD,), jnp.float32),
        pltpu.VMEM_SHARED((N_PAD,), jnp.float32),
        pltpu.VMEM_SHARED((N_PAD,), jnp.float32),
        pltpu.VMEM((6 * K_ROWS, LANES), jnp.int32),
        pltpu.VMEM((3 * K_ROWS, LANES), jnp.float32),
        pltpu.VMEM((K_ROWS, LANES), jnp.float32),
        pltpu.VMEM((STRIPE,), jnp.float32),
        pltpu.SemaphoreType.DMA,
        pltpu.SemaphoreType.DMA,
        pltpu.SemaphoreType.DMA,
        pltpu.SemaphoreType.DMA,
    ],
)(_sc_edge_body)


def _tc_pool_body(a0, a1, d0, d1, xr, br, wl, wr, bl, out_ref):
    t = pl.program_id(0)

    @pl.when(t == 0)
    def _():
        out_ref[...] = jnp.full((N_GRAPHS, OUT_F), -jnp.inf, jnp.float32)

    aggr = a0[0] + a1[0]
    deg = d0[0] + d1[0]
    mean = aggr / jnp.maximum(deg, 1.0)
    xv = xr[0]
    b = br[0]

    mcol = mean.reshape(TILE_N, 1)
    xcol = xv.reshape(TILE_N, 1)
    h = mcol * wl[...] + xcol * wr[...]

    bcol = b.reshape(TILE_N, 1)
    g_lo = jnp.minimum(b[0, 0], N_GRAPHS - 1)
    g_hi = jnp.minimum(b[0, TILE_N - 1], N_GRAPHS - 1)
    rows = lax.broadcasted_iota(jnp.int32, (N_GRAPHS, 1), 0)

    def body(g, _):
        mask = bcol == g
        row = jnp.max(jnp.where(mask, h, -jnp.inf), axis=0)
        upd = jnp.maximum(out_ref[...], row[None, :])
        out_ref[...] = jnp.where(rows == g, upd, out_ref[...])
        return 0
    lax.fori_loop(g_lo, g_hi + 1, body, 0)

    @pl.when(t == pl.num_programs(0) - 1)
    def _():
        out_ref[...] = out_ref[...] + bl[...]


def _tc_pool(a0, a1, d0, d1, xr, br, wl, wr, bl):
    node3 = lambda: pl.BlockSpec((1, 1, TILE_N), lambda t: (t, 0, 0))
    const2 = lambda: pl.BlockSpec((1, OUT_F), lambda t: (0, 0))
    return pl.pallas_call(
        _tc_pool_body,
        grid=(N_TILES,),
        in_specs=[node3(), node3(), node3(), node3(), node3(), node3(),
                  const2(), const2(), const2()],
        out_specs=pl.BlockSpec((N_GRAPHS, OUT_F), lambda t: (0, 0)),
        out_shape=jax.ShapeDtypeStruct((N_GRAPHS, OUT_F), jnp.float32),
    )(a0, a1, d0, d1, xr, br, wl, wr, bl)


def kernel(x, edge_index, batch, W_l, b_l, W_r):
    x_flat = x.reshape(N_NODES)
    x_p = jnp.pad(x_flat, (0, N_PAD - N_NODES))
    ei4 = edge_index.reshape(2, CHUNKS, K_ROWS, LANES)

    aggr_parts, deg_parts = _sc_edge_kernel(x_p, ei4)
    aggr_parts = aggr_parts.reshape(2, N_PAD)
    deg_parts = deg_parts.reshape(2, N_PAD)

    shape3 = (N_TILES, 1, TILE_N)
    a0 = aggr_parts[0].reshape(shape3)
    a1 = aggr_parts[1].reshape(shape3)
    d0 = deg_parts[0].reshape(shape3)
    d1 = deg_parts[1].reshape(shape3)
    xr = x_p.reshape(shape3)
    br = jnp.pad(batch, (0, N_PAD - N_NODES),
                 constant_values=N_GRAPHS).reshape(shape3)

    out = _tc_pool(a0, a1, d0, d1, xr, br,
                   W_l.reshape(1, OUT_F), W_r.reshape(1, OUT_F),
                   b_l.reshape(1, OUT_F))
    return out


# speedup vs baseline: 1.1416x; 1.1416x over previous
import functools

import jax
import jax.numpy as jnp
from jax import lax
from jax.experimental import pallas as pl
from jax.experimental.pallas import tpu as pltpu
from jax.experimental.pallas import tpu_sc as plsc

N_NODES = 100000
N_EDGES = 6400000
OUT_F = 128
N_GRAPHS = 64

LANES = 128
K_ROWS = 8
CHUNK_E = LANES * K_ROWS
CHUNKS = N_EDGES // CHUNK_E
NW = 32
CHUNKS_PER_W = CHUNKS // NW
CHUNKS_EXTRA = CHUNKS % NW

TILE_N = 512
N_PAD = 100352
N_TILES = N_PAD // TILE_N
STRIPE = N_PAD // 16


def _sc_edge_body(x_hbm, ei_hbm, dz_hbm, aggr_out, deg_out,
                  x_sp, aggr_sp, deg_sp,
                  idx_v, vals, ones_v, zbuf,
                  gsem, gsem2, ssem, lsem):
    c = lax.axis_index("c")
    s = lax.axis_index("s")
    wid = c * 16 + s

    def _zb(i, _):
        zbuf[pl.ds(i * 16, 16)] = jnp.zeros((16,), jnp.float32)
        return 0
    lax.fori_loop(0, STRIPE // 16, _zb, 0)
    for j in range(K_ROWS):
        for i in range(LANES // 16):
            ones_v[j, pl.ds(i * 16, 16)] = jnp.ones((16,), jnp.float32)

    st = s * STRIPE
    pltpu.async_copy(x_hbm.at[pl.ds(st, STRIPE)], x_sp.at[pl.ds(st, STRIPE)],
                     lsem).wait()
    pltpu.async_copy(zbuf, aggr_sp.at[pl.ds(st, STRIPE)], lsem).wait()
    pltpu.async_copy(zbuf, deg_sp.at[pl.ds(st, STRIPE)], lsem).wait()
    plsc.subcore_barrier()

    base = wid * CHUNKS_PER_W + jnp.minimum(wid, CHUNKS_EXTRA)
    nchunks = CHUNKS_PER_W + jnp.where(wid < CHUNKS_EXTRA, 1, 0)

    def vdrain(sem, n):
        for j in range(n):
            pltpu.make_async_copy(dz_hbm, vals.at[0, j], sem).wait()

    def ldrain(n):
        for j in range(n):
            pltpu.make_async_copy(ei_hbm.at[:, pl.ds(0, LANES)],
                                  idx_v.at[0, j], lsem).wait()

    def load_chunk(ck, pset):
        for j in range(K_ROWS):
            off = ck * CHUNK_E + j * LANES
            pltpu.async_copy(ei_hbm.at[:, pl.ds(off, LANES)],
                             idx_v.at[pset, j], lsem)

    load_chunk(base, 0)

    def round_body(r, _):
        p = lax.rem(r, 3)
        pm1 = lax.rem(r + 2, 3)

        @pl.when(jnp.logical_or(r == 1, r == nchunks + 1))
        def _():
            vdrain(ssem, K_ROWS)

        @pl.when(jnp.logical_and(r >= 2, r <= nchunks))
        def _():
            vdrain(ssem, 2 * K_ROWS)

        @pl.when(r < nchunks)
        def _():
            ldrain(K_ROWS)

            @pl.when(r + 1 < nchunks)
            def _():
                load_chunk(base + r + 1, lax.rem(r + 1, 3))

            for j in range(K_ROWS):
                pltpu.async_copy(ones_v.at[j], deg_sp.at[idx_v.at[p, j, 1]],
                                 ssem, add=True)

            @pl.when(lax.rem(r, 2) == 0)
            def _():
                for j in range(K_ROWS):
                    pltpu.async_copy(x_sp.at[idx_v.at[p, j, 0]],
                                     vals.at[p, j], gsem)

            @pl.when(lax.rem(r, 2) == 1)
            def _():
                for j in range(K_ROWS):
                    pltpu.async_copy(x_sp.at[idx_v.at[p, j, 0]],
                                     vals.at[p, j], gsem2)

        @pl.when(jnp.logical_and(r >= 1, r <= nchunks))
        def _():
            @pl.when(lax.rem(r, 2) == 1)
            def _():
                vdrain(gsem, K_ROWS)

            @pl.when(lax.rem(r, 2) == 0)
            def _():
                vdrain(gsem2, K_ROWS)

            for j in range(K_ROWS):
                pltpu.async_copy(vals.at[pm1, j],
                                 aggr_sp.at[idx_v.at[pm1, j, 1]],
                                 ssem, add=True)
        return 0
    lax.fori_loop(0, nchunks + 2, round_body, 0)

    plsc.subcore_barrier()

    pltpu.async_copy(aggr_sp.at[pl.ds(st, STRIPE)],
                     aggr_out.at[pl.ds(c * N_PAD + st, STRIPE)], lsem).wait()
    pltpu.async_copy(deg_sp.at[pl.ds(st, STRIPE)],
                     deg_out.at[pl.ds(c * N_PAD + st, STRIPE)], lsem).wait()


_sc_edge_kernel = functools.partial(
    pl.kernel,
    out_type=[jax.ShapeDtypeStruct((2 * N_PAD,), jnp.float32),
              jax.ShapeDtypeStruct((2 * N_PAD,), jnp.float32)],
    mesh=plsc.VectorSubcoreMesh(core_axis_name="c", subcore_axis_name="s"),
    scratch_types=[
        pltpu.VMEM_SHARED((N_PAD,), jnp.float32),
        pltpu.VMEM_SHARED((N_PAD,), jnp.float32),
        pltpu.VMEM_SHARED((N_PAD,), jnp.float32),
        pltpu.VMEM((3, K_ROWS, 2, LANES), jnp.int32),
        pltpu.VMEM((3, K_ROWS, LANES), jnp.float32),
        pltpu.VMEM((K_ROWS, LANES), jnp.float32),
        pltpu.VMEM((STRIPE,), jnp.float32),
        pltpu.SemaphoreType.DMA,
        pltpu.SemaphoreType.DMA,
        pltpu.SemaphoreType.DMA,
        pltpu.SemaphoreType.DMA,
    ],
)(_sc_edge_body)


def _tc_pool_body(a0, a1, d0, d1, xr, br, wl, wr, bl, out_ref):
    t = pl.program_id(0)

    @pl.when(t == 0)
    def _():
        out_ref[...] = jnp.full((N_GRAPHS, OUT_F), -jnp.inf, jnp.float32)

    aggr = a0[0] + a1[0]
    deg = d0[0] + d1[0]
    mean = aggr / jnp.maximum(deg, 1.0)
    xv = xr[0]
    b = br[0]

    mcol = mean.reshape(TILE_N, 1)
    xcol = xv.reshape(TILE_N, 1)
    h = mcol * wl[...] + xcol * wr[...]

    bcol = b.reshape(TILE_N, 1)
    g_lo = jnp.minimum(b[0, 0], N_GRAPHS - 1)
    g_hi = jnp.minimum(b[0, TILE_N - 1], N_GRAPHS - 1)
    rows = lax.broadcasted_iota(jnp.int32, (N_GRAPHS, 1), 0)

    def body(g, _):
        mask = bcol == g
        row = jnp.max(jnp.where(mask, h, -jnp.inf), axis=0)
        upd = jnp.maximum(out_ref[...], row[None, :])
        out_ref[...] = jnp.where(rows == g, upd, out_ref[...])
        return 0
    lax.fori_loop(g_lo, g_hi + 1, body, 0)

    @pl.when(t == pl.num_programs(0) - 1)
    def _():
        out_ref[...] = out_ref[...] + bl[...]


def _tc_pool(a0, a1, d0, d1, xr, br, wl, wr, bl):
    node3 = lambda: pl.BlockSpec((1, 1, TILE_N), lambda t: (t, 0, 0))
    const2 = lambda: pl.BlockSpec((1, OUT_F), lambda t: (0, 0))
    return pl.pallas_call(
        _tc_pool_body,
        grid=(N_TILES,),
        in_specs=[node3(), node3(), node3(), node3(), node3(), node3(),
                  const2(), const2(), const2()],
        out_specs=pl.BlockSpec((N_GRAPHS, OUT_F), lambda t: (0, 0)),
        out_shape=jax.ShapeDtypeStruct((N_GRAPHS, OUT_F), jnp.float32),
    )(a0, a1, d0, d1, xr, br, wl, wr, bl)


def kernel(x, edge_index, batch, W_l, b_l, W_r):
    x_flat = x.reshape(N_NODES)
    x_p = jnp.pad(x_flat, (0, N_PAD - N_NODES))

    dz = jnp.zeros((LANES,), jnp.float32)
    aggr_parts, deg_parts = _sc_edge_kernel(x_p, edge_index, dz)
    aggr_parts = aggr_parts.reshape(2, N_PAD)
    deg_parts = deg_parts.reshape(2, N_PAD)

    shape3 = (N_TILES, 1, TILE_N)
    a0 = aggr_parts[0].reshape(shape3)
    a1 = aggr_parts[1].reshape(shape3)
    d0 = deg_parts[0].reshape(shape3)
    d1 = deg_parts[1].reshape(shape3)
    xr = x_p.reshape(shape3)
    br = jnp.pad(batch, (0, N_PAD - N_NODES),
                 constant_values=N_GRAPHS).reshape(shape3)

    out = _tc_pool(a0, a1, d0, d1, xr, br,
                   W_l.reshape(1, OUT_F), W_r.reshape(1, OUT_F),
                   b_l.reshape(1, OUT_F))
    return out


# speedup vs baseline: 1.1487x; 1.0063x over previous
import functools

import jax
import jax.numpy as jnp
from jax import lax
from jax.experimental import pallas as pl
from jax.experimental.pallas import tpu as pltpu
from jax.experimental.pallas import tpu_sc as plsc

N_NODES = 100000
N_EDGES = 6400000
OUT_F = 128
N_GRAPHS = 64

LANES = 128
K_ROWS = 8
CHUNK_E = LANES * K_ROWS
CHUNKS = N_EDGES // CHUNK_E
NW = 32
CHUNKS_PER_W = CHUNKS // NW
CHUNKS_EXTRA = CHUNKS % NW

TILE_N = 512
N_PAD = 100352
N_TILES = N_PAD // TILE_N
STRIPE = N_PAD // 16


def _sc_edge_body(x_hbm, ei_hbm, dz_hbm, aggr_out, deg_out,
                  x_sp, aggr_sp, deg_sp,
                  idx_v, vals, ones_v, zbuf,
                  gsem, gsem2, ssem, lsem):
    c = lax.axis_index("c")
    s = lax.axis_index("s")
    wid = c * 16 + s

    def _zb(i, _):
        zbuf[pl.ds(i * 16, 16)] = jnp.zeros((16,), jnp.float32)
        return 0
    lax.fori_loop(0, STRIPE // 16, _zb, 0)
    for j in range(K_ROWS):
        for i in range(LANES // 16):
            ones_v[j, pl.ds(i * 16, 16)] = jnp.ones((16,), jnp.float32)

    st = s * STRIPE
    pltpu.async_copy(x_hbm.at[pl.ds(st, STRIPE)], x_sp.at[pl.ds(st, STRIPE)],
                     lsem).wait()
    pltpu.async_copy(zbuf, aggr_sp.at[pl.ds(st, STRIPE)], lsem).wait()
    pltpu.async_copy(zbuf, deg_sp.at[pl.ds(st, STRIPE)], lsem).wait()
    plsc.subcore_barrier()

    base = wid * CHUNKS_PER_W + jnp.minimum(wid, CHUNKS_EXTRA)
    nchunks = CHUNKS_PER_W + jnp.where(wid < CHUNKS_EXTRA, 1, 0)

    def vdrain(sem, n):
        for j in range(n):
            pltpu.make_async_copy(dz_hbm, vals.at[0, j], sem).wait()

    def ldrain(n):
        for j in range(n):
            pltpu.make_async_copy(ei_hbm.at[:, pl.ds(0, LANES)],
                                  idx_v.at[0, j], lsem).wait()

    def load_chunk(ck, pset):
        for j in range(K_ROWS):
            off = ck * CHUNK_E + j * LANES
            pltpu.async_copy(ei_hbm.at[:, pl.ds(off, LANES)],
                             idx_v.at[pset, j], lsem)

    load_chunk(base, 0)

    def round_body(r, _):
        p = lax.rem(r, 3)
        pm1 = lax.rem(r + 2, 3)

        @pl.when(jnp.logical_or(r == 1, r == nchunks + 1))
        def _():
            vdrain(ssem, K_ROWS)

        @pl.when(jnp.logical_and(r >= 2, r <= nchunks))
        def _():
            vdrain(ssem, 2 * K_ROWS)

        @pl.when(r < nchunks)
        def _():
            ldrain(K_ROWS)

            @pl.when(r + 1 < nchunks)
            def _():
                load_chunk(base + r + 1, lax.rem(r + 1, 3))

            for j in range(K_ROWS):
                pltpu.async_copy(ones_v.at[j], deg_sp.at[idx_v.at[p, j, 1]],
                                 ssem, add=True)

            @pl.when(lax.rem(r, 2) == 0)
            def _():
                for j in range(K_ROWS):
                    pltpu.async_copy(x_sp.at[idx_v.at[p, j, 0]],
                                     vals.at[p, j], gsem)

            @pl.when(lax.rem(r, 2) == 1)
            def _():
                for j in range(K_ROWS):
                    pltpu.async_copy(x_sp.at[idx_v.at[p, j, 0]],
                                     vals.at[p, j], gsem2)

        @pl.when(jnp.logical_and(r >= 1, r <= nchunks))
        def _():
            @pl.when(lax.rem(r, 2) == 1)
            def _():
                vdrain(gsem, K_ROWS)

            @pl.when(lax.rem(r, 2) == 0)
            def _():
                vdrain(gsem2, K_ROWS)

            for j in range(K_ROWS):
                pltpu.async_copy(vals.at[pm1, j],
                                 aggr_sp.at[idx_v.at[pm1, j, 1]],
                                 ssem, add=True)
        return 0
    lax.fori_loop(0, nchunks + 2, round_body, 0)

    plsc.subcore_barrier()

    pltpu.async_copy(aggr_sp.at[pl.ds(st, STRIPE)],
                     aggr_out.at[pl.ds(c * N_PAD + st, STRIPE)], lsem).wait()
    pltpu.async_copy(deg_sp.at[pl.ds(st, STRIPE)],
                     deg_out.at[pl.ds(c * N_PAD + st, STRIPE)], lsem).wait()


_sc_edge_kernel = functools.partial(
    pl.kernel,
    out_type=[jax.ShapeDtypeStruct((2 * N_PAD,), jnp.float32),
              jax.ShapeDtypeStruct((2 * N_PAD,), jnp.float32)],
    mesh=plsc.VectorSubcoreMesh(core_axis_name="c", subcore_axis_name="s"),
    scratch_types=[
        pltpu.VMEM_SHARED((N_PAD,), jnp.float32),
        pltpu.VMEM_SHARED((N_PAD,), jnp.float32),
        pltpu.VMEM_SHARED((N_PAD,), jnp.float32),
        pltpu.VMEM((3, K_ROWS, 2, LANES), jnp.int32),
        pltpu.VMEM((3, K_ROWS, LANES), jnp.float32),
        pltpu.VMEM((K_ROWS, LANES), jnp.float32),
        pltpu.VMEM((STRIPE,), jnp.float32),
        pltpu.SemaphoreType.DMA,
        pltpu.SemaphoreType.DMA,
        pltpu.SemaphoreType.DMA,
        pltpu.SemaphoreType.DMA,
    ],
)(_sc_edge_body)


def _tc_pool_body(ap, dp, xr, br, wl, wr, bl, out_ref):
    t = pl.program_id(0)

    @pl.when(t == 0)
    def _():
        out_ref[...] = jnp.full((N_GRAPHS, OUT_F), -jnp.inf, jnp.float32)

    aggr = ap[0, 0] + ap[1, 0]
    deg = dp[0, 0] + dp[1, 0]
    mean = aggr / jnp.maximum(deg, 1.0)
    xv = xr[0]
    b = br[0]

    mcol = mean.reshape(TILE_N, 1)
    xcol = xv.reshape(TILE_N, 1)
    h = mcol * wl[...] + xcol * wr[...]

    bcol = b.reshape(TILE_N, 1)
    g_lo = jnp.minimum(b[0, 0], N_GRAPHS - 1)
    g_hi = jnp.minimum(b[0, TILE_N - 1], N_GRAPHS - 1)
    rows = lax.broadcasted_iota(jnp.int32, (N_GRAPHS, 1), 0)

    def body(g, _):
        mask = bcol == g
        row = jnp.max(jnp.where(mask, h, -jnp.inf), axis=0)
        upd = jnp.maximum(out_ref[...], row[None, :])
        out_ref[...] = jnp.where(rows == g, upd, out_ref[...])
        return 0
    lax.fori_loop(g_lo, g_hi + 1, body, 0)

    @pl.when(t == pl.num_programs(0) - 1)
    def _():
        out_ref[...] = out_ref[...] + bl[...]


def _tc_pool(ap, dp, xr, br, wl, wr, bl):
    part4 = lambda: pl.BlockSpec((2, 1, 1, TILE_N), lambda t: (0, t, 0, 0))
    node3 = lambda: pl.BlockSpec((1, 1, TILE_N), lambda t: (t, 0, 0))
    const2 = lambda: pl.BlockSpec((1, OUT_F), lambda t: (0, 0))
    return pl.pallas_call(
        _tc_pool_body,
        grid=(N_TILES,),
        in_specs=[part4(), part4(), node3(), node3(),
                  const2(), const2(), const2()],
        out_specs=pl.BlockSpec((N_GRAPHS, OUT_F), lambda t: (0, 0)),
        out_shape=jax.ShapeDtypeStruct((N_GRAPHS, OUT_F), jnp.float32),
    )(ap, dp, xr, br, wl, wr, bl)


def kernel(x, edge_index, batch, W_l, b_l, W_r):
    x_flat = x.reshape(N_NODES)
    x_p = jnp.pad(x_flat, (0, N_PAD - N_NODES))

    dz = jnp.zeros((LANES,), jnp.float32)
    aggr_parts, deg_parts = _sc_edge_kernel(x_p, edge_index, dz)

    shape4 = (2, N_TILES, 1, TILE_N)
    ap = aggr_parts.reshape(shape4)
    dp = deg_parts.reshape(shape4)
    shape3 = (N_TILES, 1, TILE_N)
    xr = x_p.reshape(shape3)
    br = jnp.pad(batch, (0, N_PAD - N_NODES),
                 constant_values=N_GRAPHS).reshape(shape3)

    out = _tc_pool(ap, dp, xr, br,
                   W_l.reshape(1, OUT_F), W_r.reshape(1, OUT_F),
                   b_l.reshape(1, OUT_F))
    return out


# speedup vs baseline: 1.4589x; 1.2700x over previous
import functools

import jax
import jax.numpy as jnp
from jax import lax
from jax.experimental import pallas as pl
from jax.experimental.pallas import tpu as pltpu
from jax.experimental.pallas import tpu_sc as plsc

N_NODES = 100000
N_EDGES = 6400000
OUT_F = 128
N_GRAPHS = 64

LANES = 128
K_ROWS = 8
CHUNK_E = LANES * K_ROWS
CHUNKS = N_EDGES // CHUNK_E
NW = 32
CHUNKS_PER_W = CHUNKS // NW
CHUNKS_EXTRA = CHUNKS % NW

TILE_N = 512
N_PAD = 100352
N_TILES = N_PAD // TILE_N
STRIPE = N_PAD // 16
ZB = STRIPE // 8


def _sc_edge_body(x_hbm, ei_hbm, dz_hbm, aggr_out, deg_out,
                  aggr_sp, deg_sp,
                  x_tile, idx_v, vals, ones_v, zbuf,
                  ssem, lsem):
    c = lax.axis_index("c")
    s = lax.axis_index("s")
    wid = c * 16 + s

    def _zb(i, _):
        zbuf[pl.ds(i * 16, 16)] = jnp.zeros((16,), jnp.float32)
        return 0
    lax.fori_loop(0, ZB // 16, _zb, 0)
    for j in range(K_ROWS):
        for i in range(LANES // 16):
            ones_v[j, pl.ds(i * 16, 16)] = jnp.ones((16,), jnp.float32)

    st = s * STRIPE
    pltpu.async_copy(x_hbm, x_tile, lsem).wait()
    for q in range(8):
        pltpu.async_copy(zbuf, aggr_sp.at[pl.ds(st + q * ZB, ZB)], lsem).wait()
        pltpu.async_copy(zbuf, deg_sp.at[pl.ds(st + q * ZB, ZB)], lsem).wait()
    plsc.subcore_barrier()

    base = wid * CHUNKS_PER_W + jnp.minimum(wid, CHUNKS_EXTRA)
    nchunks = CHUNKS_PER_W + jnp.where(wid < CHUNKS_EXTRA, 1, 0)

    def vdrain(sem, n):
        for j in range(n):
            pltpu.make_async_copy(dz_hbm, vals.at[0, j], sem).wait()

    def ldrain(n):
        for j in range(n):
            pltpu.make_async_copy(ei_hbm.at[:, pl.ds(0, LANES)],
                                  idx_v.at[0, j], lsem).wait()

    def load_chunk(ck, pset):
        for j in range(K_ROWS):
            off = ck * CHUNK_E + j * LANES
            pltpu.async_copy(ei_hbm.at[:, pl.ds(off, LANES)],
                             idx_v.at[pset, j], lsem)

    load_chunk(base, 0)

    def round_body(r, _):
        p = lax.rem(r, 3)

        @pl.when(jnp.logical_and(r >= 1, r <= nchunks))
        def _():
            vdrain(ssem, 2 * K_ROWS)

        @pl.when(r < nchunks)
        def _():
            ldrain(K_ROWS)

            @pl.when(r + 1 < nchunks)
            def _():
                load_chunk(base + r + 1, lax.rem(r + 1, 3))

            for j in range(K_ROWS):
                pltpu.async_copy(ones_v.at[j], deg_sp.at[idx_v.at[p, j, 1]],
                                 ssem, add=True)

            for j in range(K_ROWS):
                for i in range(LANES // 16):
                    vidx = idx_v[p, j, 0, pl.ds(i * 16, 16)]
                    vals[p, j, pl.ds(i * 16, 16)] = plsc.load_gather(
                        x_tile, [vidx])

            for j in range(K_ROWS):
                pltpu.async_copy(vals.at[p, j],
                                 aggr_sp.at[idx_v.at[p, j, 1]],
                                 ssem, add=True)
        return 0
    lax.fori_loop(0, nchunks + 1, round_body, 0)

    plsc.subcore_barrier()

    pltpu.async_copy(aggr_sp.at[pl.ds(st, STRIPE)],
                     aggr_out.at[pl.ds(c * N_PAD + st, STRIPE)], lsem).wait()
    pltpu.async_copy(deg_sp.at[pl.ds(st, STRIPE)],
                     deg_out.at[pl.ds(c * N_PAD + st, STRIPE)], lsem).wait()


_sc_edge_kernel = functools.partial(
    pl.kernel,
    out_type=[jax.ShapeDtypeStruct((2 * N_PAD,), jnp.float32),
              jax.ShapeDtypeStruct((2 * N_PAD,), jnp.float32)],
    mesh=plsc.VectorSubcoreMesh(core_axis_name="c", subcore_axis_name="s"),
    compiler_params=pltpu.CompilerParams(needs_layout_passes=False),
    scratch_types=[
        pltpu.VMEM_SHARED((N_PAD,), jnp.float32),
        pltpu.VMEM_SHARED((N_PAD,), jnp.float32),
        pltpu.VMEM((N_PAD,), jnp.float32),
        pltpu.VMEM((3, K_ROWS, 2, LANES), jnp.int32),
        pltpu.VMEM((3, K_ROWS, LANES), jnp.float32),
        pltpu.VMEM((K_ROWS, LANES), jnp.float32),
        pltpu.VMEM((ZB,), jnp.float32),
        pltpu.SemaphoreType.DMA,
        pltpu.SemaphoreType.DMA,
    ],
)(_sc_edge_body)


def _tc_pool_body(ap, dp, xr, br, wl, wr, bl, out_ref):
    t = pl.program_id(0)

    @pl.when(t == 0)
    def _():
        out_ref[...] = jnp.full((N_GRAPHS, OUT_F), -jnp.inf, jnp.float32)

    aggr = ap[0, 0] + ap[1, 0]
    deg = dp[0, 0] + dp[1, 0]
    mean = aggr / jnp.maximum(deg, 1.0)
    xv = xr[0]
    b = br[0]

    mcol = mean.reshape(TILE_N, 1)
    xcol = xv.reshape(TILE_N, 1)
    h = mcol * wl[...] + xcol * wr[...]

    bcol = b.reshape(TILE_N, 1)
    g_lo = jnp.minimum(b[0, 0], N_GRAPHS - 1)
    g_hi = jnp.minimum(b[0, TILE_N - 1], N_GRAPHS - 1)
    rows = lax.broadcasted_iota(jnp.int32, (N_GRAPHS, 1), 0)

    def body(g, _):
        mask = bcol == g
        row = jnp.max(jnp.where(mask, h, -jnp.inf), axis=0)
        upd = jnp.maximum(out_ref[...], row[None, :])
        out_ref[...] = jnp.where(rows == g, upd, out_ref[...])
        return 0
    lax.fori_loop(g_lo, g_hi + 1, body, 0)

    @pl.when(t == pl.num_programs(0) - 1)
    def _():
        out_ref[...] = out_ref[...] + bl[...]


def _tc_pool(ap, dp, xr, br, wl, wr, bl):
    part4 = lambda: pl.BlockSpec((2, 1, 1, TILE_N), lambda t: (0, t, 0, 0))
    node3 = lambda: pl.BlockSpec((1, 1, TILE_N), lambda t: (t, 0, 0))
    const2 = lambda: pl.BlockSpec((1, OUT_F), lambda t: (0, 0))
    return pl.pallas_call(
        _tc_pool_body,
        grid=(N_TILES,),
        in_specs=[part4(), part4(), node3(), node3(),
                  const2(), const2(), const2()],
        out_specs=pl.BlockSpec((N_GRAPHS, OUT_F), lambda t: (0, 0)),
        out_shape=jax.ShapeDtypeStruct((N_GRAPHS, OUT_F), jnp.float32),
    )(ap, dp, xr, br, wl, wr, bl)


def kernel(x, edge_index, batch, W_l, b_l, W_r):
    x_flat = x.reshape(N_NODES)
    x_p = jnp.pad(x_flat, (0, N_PAD - N_NODES))

    dz = jnp.zeros((LANES,), jnp.float32)
    aggr_parts, deg_parts = _sc_edge_kernel(x_p, edge_index, dz)

    shape4 = (2, N_TILES, 1, TILE_N)
    ap = aggr_parts.reshape(shape4)
    dp = deg_parts.reshape(shape4)
    shape3 = (N_TILES, 1, TILE_N)
    xr = x_p.reshape(shape3)
    br = jnp.pad(batch, (0, N_PAD - N_NODES),
                 constant_values=N_GRAPHS).reshape(shape3)

    out = _tc_pool(ap, dp, xr, br,
                   W_l.reshape(1, OUT_F), W_r.reshape(1, OUT_F),
                   b_l.reshape(1, OUT_F))
    return out


# speedup vs baseline: 1.6299x; 1.1172x over previous
import functools

import jax
import jax.numpy as jnp
from jax import lax
from jax.experimental import pallas as pl
from jax.experimental.pallas import tpu as pltpu
from jax.experimental.pallas import tpu_sc as plsc

N_NODES = 100000
N_EDGES = 6400000
OUT_F = 128
N_GRAPHS = 64

LANES = 128
K_ROWS = 8
CHUNK_E = LANES * K_ROWS
CHUNKS = N_EDGES // CHUNK_E
NW = 32
CHUNKS_PER_W = CHUNKS // NW
CHUNKS_EXTRA = CHUNKS % NW

TILE_N = 2048
N_PAD = 100352
N_TILES = N_PAD // TILE_N
STRIPE = N_PAD // 16
ZB = STRIPE // 8


def _sc_edge_body(x_hbm, ei_hbm, dz_hbm, aggr_out, deg_out,
                  aggr_sp, deg_sp,
                  x_tile, idx_v, vals, ones_v, zbuf,
                  ssem, lsem):
    c = lax.axis_index("c")
    s = lax.axis_index("s")
    wid = c * 16 + s

    def _zb(i, _):
        zbuf[pl.ds(i * 16, 16)] = jnp.zeros((16,), jnp.float32)
        return 0
    lax.fori_loop(0, ZB // 16, _zb, 0)
    for j in range(K_ROWS):
        for i in range(LANES // 16):
            ones_v[j, pl.ds(i * 16, 16)] = jnp.ones((16,), jnp.float32)

    st = s * STRIPE
    pltpu.async_copy(x_hbm, x_tile, lsem).wait()
    for q in range(8):
        pltpu.async_copy(zbuf, aggr_sp.at[pl.ds(st + q * ZB, ZB)], lsem).wait()
        pltpu.async_copy(zbuf, deg_sp.at[pl.ds(st + q * ZB, ZB)], lsem).wait()
    plsc.subcore_barrier()

    base = wid * CHUNKS_PER_W + jnp.minimum(wid, CHUNKS_EXTRA)
    nchunks = CHUNKS_PER_W + jnp.where(wid < CHUNKS_EXTRA, 1, 0)

    def vdrain(sem, n):
        for j in range(n):
            pltpu.make_async_copy(dz_hbm, vals.at[0, j], sem).wait()

    def ldrain(n):
        for j in range(n):
            pltpu.make_async_copy(ei_hbm.at[:, pl.ds(0, LANES)],
                                  idx_v.at[0, j], lsem).wait()

    def load_chunk(ck, pset):
        for j in range(K_ROWS):
            off = ck * CHUNK_E + j * LANES
            pltpu.async_copy(ei_hbm.at[:, pl.ds(off, LANES)],
                             idx_v.at[pset, j], lsem)

    load_chunk(base, 0)

    def round_body(r, _):
        p = lax.rem(r, 3)

        @pl.when(jnp.logical_and(r >= 1, r <= nchunks))
        def _():
            vdrain(ssem, 2 * K_ROWS)

        @pl.when(r < nchunks)
        def _():
            ldrain(K_ROWS)

            @pl.when(r + 1 < nchunks)
            def _():
                load_chunk(base + r + 1, lax.rem(r + 1, 3))

            for j in range(K_ROWS):
                pltpu.async_copy(ones_v.at[j], deg_sp.at[idx_v.at[p, j, 1]],
                                 ssem, add=True)

            for j in range(K_ROWS):
                for i in range(LANES // 16):
                    vidx = idx_v[p, j, 0, pl.ds(i * 16, 16)]
                    vals[p, j, pl.ds(i * 16, 16)] = plsc.load_gather(
                        x_tile, [vidx])

            for j in range(K_ROWS):
                pltpu.async_copy(vals.at[p, j],
                                 aggr_sp.at[idx_v.at[p, j, 1]],
                                 ssem, add=True)
        return 0
    lax.fori_loop(0, nchunks + 1, round_body, 0)

    plsc.subcore_barrier()

    pltpu.async_copy(aggr_sp.at[pl.ds(st, STRIPE)],
                     aggr_out.at[pl.ds(c * N_PAD + st, STRIPE)], lsem).wait()
    pltpu.async_copy(deg_sp.at[pl.ds(st, STRIPE)],
                     deg_out.at[pl.ds(c * N_PAD + st, STRIPE)], lsem).wait()


_sc_edge_kernel = functools.partial(
    pl.kernel,
    out_type=[jax.ShapeDtypeStruct((2 * N_PAD,), jnp.float32),
              jax.ShapeDtypeStruct((2 * N_PAD,), jnp.float32)],
    mesh=plsc.VectorSubcoreMesh(core_axis_name="c", subcore_axis_name="s"),
    compiler_params=pltpu.CompilerParams(needs_layout_passes=False),
    scratch_types=[
        pltpu.VMEM_SHARED((N_PAD,), jnp.float32),
        pltpu.VMEM_SHARED((N_PAD,), jnp.float32),
        pltpu.VMEM((N_PAD,), jnp.float32),
        pltpu.VMEM((3, K_ROWS, 2, LANES), jnp.int32),
        pltpu.VMEM((3, K_ROWS, LANES), jnp.float32),
        pltpu.VMEM((K_ROWS, LANES), jnp.float32),
        pltpu.VMEM((ZB,), jnp.float32),
        pltpu.SemaphoreType.DMA,
        pltpu.SemaphoreType.DMA,
    ],
)(_sc_edge_body)


def _tc_pool_body(ap, dp, xr, br, wl, wr, bl, out_ref):
    t = pl.program_id(0)

    @pl.when(t == 0)
    def _():
        out_ref[...] = jnp.full((N_GRAPHS, OUT_F), -jnp.inf, jnp.float32)

    aggr = ap[0, 0] + ap[1, 0]
    deg = dp[0, 0] + dp[1, 0]
    mean = aggr / jnp.maximum(deg, 1.0)
    xv = xr[0]
    b = br[0]

    mcol = mean.reshape(TILE_N, 1)
    xcol = xv.reshape(TILE_N, 1)
    h = mcol * wl[...] + xcol * wr[...]

    bcol = b.reshape(TILE_N, 1)
    g_lo = jnp.minimum(b[0, 0], N_GRAPHS - 1)
    g_hi = jnp.minimum(b[0, TILE_N - 1], N_GRAPHS - 1)
    rows = lax.broadcasted_iota(jnp.int32, (N_GRAPHS, 1), 0)

    def body(g, _):
        mask = bcol == g
        row = jnp.max(jnp.where(mask, h, -jnp.inf), axis=0)
        upd = jnp.maximum(out_ref[...], row[None, :])
        out_ref[...] = jnp.where(rows == g, upd, out_ref[...])
        return 0
    lax.fori_loop(g_lo, g_hi + 1, body, 0)

    @pl.when(t == pl.num_programs(0) - 1)
    def _():
        out_ref[...] = out_ref[...] + bl[...]


def _tc_pool(ap, dp, xr, br, wl, wr, bl):
    part4 = lambda: pl.BlockSpec((2, 1, 1, TILE_N), lambda t: (0, t, 0, 0))
    node3 = lambda: pl.BlockSpec((1, 1, TILE_N), lambda t: (t, 0, 0))
    const2 = lambda: pl.BlockSpec((1, OUT_F), lambda t: (0, 0))
    return pl.pallas_call(
        _tc_pool_body,
        grid=(N_TILES,),
        in_specs=[part4(), part4(), node3(), node3(),
                  const2(), const2(), const2()],
        out_specs=pl.BlockSpec((N_GRAPHS, OUT_F), lambda t: (0, 0)),
        out_shape=jax.ShapeDtypeStruct((N_GRAPHS, OUT_F), jnp.float32),
    )(ap, dp, xr, br, wl, wr, bl)


def kernel(x, edge_index, batch, W_l, b_l, W_r):
    x_flat = x.reshape(N_NODES)
    x_p = jnp.pad(x_flat, (0, N_PAD - N_NODES))

    dz = jnp.zeros((LANES,), jnp.float32)
    aggr_parts, deg_parts = _sc_edge_kernel(x_p, edge_index, dz)

    shape4 = (2, N_TILES, 1, TILE_N)
    ap = aggr_parts.reshape(shape4)
    dp = deg_parts.reshape(shape4)
    shape3 = (N_TILES, 1, TILE_N)
    xr = x_p.reshape(shape3)
    br = jnp.pad(batch, (0, N_PAD - N_NODES),
                 constant_values=N_GRAPHS).reshape(shape3)

    out = _tc_pool(ap, dp, xr, br,
                   W_l.reshape(1, OUT_F), W_r.reshape(1, OUT_F),
                   b_l.reshape(1, OUT_F))
    return out


# speedup vs baseline: 1.6356x; 1.0035x over previous
import functools

import jax
import jax.numpy as jnp
from jax import lax
from jax.experimental import pallas as pl
from jax.experimental.pallas import tpu as pltpu
from jax.experimental.pallas import tpu_sc as plsc

N_NODES = 100000
N_EDGES = 6400000
OUT_F = 128
N_GRAPHS = 64

LANES = 128
K_ROWS = 8
CHUNK_E = LANES * K_ROWS
CHUNKS = N_EDGES // CHUNK_E
NW = 32
CHUNKS_PER_W = CHUNKS // NW
CHUNKS_EXTRA = CHUNKS % NW

TILE_N = 2048
N_PAD = 100352
N_TILES = N_PAD // TILE_N
STRIPE = N_PAD // 16
ZB = STRIPE // 8


def _sc_edge_body(x_hbm, ei_hbm, dz_hbm, aggr_out, deg_out,
                  aggr_sp, deg_sp,
                  x_tile, idx_v, vals, ones_v, zbuf,
                  ssem, lsem):
    c = lax.axis_index("c")
    s = lax.axis_index("s")
    wid = c * 16 + s

    def _zb(i, _):
        zbuf[pl.ds(i * 16, 16)] = jnp.zeros((16,), jnp.float32)
        return 0
    lax.fori_loop(0, ZB // 16, _zb, 0)
    for j in range(K_ROWS):
        for i in range(LANES // 16):
            ones_v[j, pl.ds(i * 16, 16)] = jnp.ones((16,), jnp.float32)

    st = s * STRIPE
    pltpu.async_copy(x_hbm, x_tile, lsem).wait()
    for q in range(8):
        pltpu.async_copy(zbuf, aggr_sp.at[pl.ds(st + q * ZB, ZB)], lsem).wait()
        pltpu.async_copy(zbuf, deg_sp.at[pl.ds(st + q * ZB, ZB)], lsem).wait()
    plsc.subcore_barrier()

    base = wid * CHUNKS_PER_W + jnp.minimum(wid, CHUNKS_EXTRA)
    nchunks = CHUNKS_PER_W + jnp.where(wid < CHUNKS_EXTRA, 1, 0)

    def vdrain(sem, n):
        for j in range(n):
            pltpu.make_async_copy(dz_hbm, vals.at[0, j], sem).wait()

    def ldrain(n):
        for j in range(n):
            pltpu.make_async_copy(ei_hbm.at[:, pl.ds(0, LANES)],
                                  idx_v.at[0, j], lsem).wait()

    def load_chunk(ck, pset):
        for j in range(K_ROWS):
            off = ck * CHUNK_E + j * LANES
            pltpu.async_copy(ei_hbm.at[:, pl.ds(off, LANES)],
                             idx_v.at[pset, j], lsem)

    load_chunk(base, 0)

    def round_body(r, _):
        p = lax.rem(r, 3)

        @pl.when(jnp.logical_and(r >= 1, r <= nchunks))
        def _():
            vdrain(ssem, 2 * K_ROWS)

        @pl.when(r < nchunks)
        def _():
            ldrain(K_ROWS)

            @pl.when(r + 1 < nchunks)
            def _():
                load_chunk(base + r + 1, lax.rem(r + 1, 3))

            for j in range(K_ROWS):
                pltpu.async_copy(ones_v.at[j], deg_sp.at[idx_v.at[p, j, 1]],
                                 ssem, add=True)

            for j in range(K_ROWS):
                for i in range(LANES // 16):
                    vidx = idx_v[p, j, 0, pl.ds(i * 16, 16)]
                    vals[p, j, pl.ds(i * 16, 16)] = plsc.load_gather(
                        x_tile, [vidx])

            for j in range(K_ROWS):
                pltpu.async_copy(vals.at[p, j],
                                 aggr_sp.at[idx_v.at[p, j, 1]],
                                 ssem, add=True)
        return 0
    lax.fori_loop(0, nchunks + 1, round_body, 0)

    plsc.subcore_barrier()

    pltpu.async_copy(aggr_sp.at[pl.ds(st, STRIPE)],
                     aggr_out.at[pl.ds(c * N_PAD + st, STRIPE)], lsem).wait()
    pltpu.async_copy(deg_sp.at[pl.ds(st, STRIPE)],
                     deg_out.at[pl.ds(c * N_PAD + st, STRIPE)], lsem).wait()


_sc_edge_kernel = functools.partial(
    pl.kernel,
    out_type=[jax.ShapeDtypeStruct((2 * N_PAD,), jnp.float32),
              jax.ShapeDtypeStruct((2 * N_PAD,), jnp.float32)],
    mesh=plsc.VectorSubcoreMesh(core_axis_name="c", subcore_axis_name="s"),
    compiler_params=pltpu.CompilerParams(needs_layout_passes=False),
    scratch_types=[
        pltpu.VMEM_SHARED((N_PAD,), jnp.float32),
        pltpu.VMEM_SHARED((N_PAD,), jnp.float32),
        pltpu.VMEM((N_PAD,), jnp.float32),
        pltpu.VMEM((3, K_ROWS, 2, LANES), jnp.int32),
        pltpu.VMEM((3, K_ROWS, LANES), jnp.float32),
        pltpu.VMEM((K_ROWS, LANES), jnp.float32),
        pltpu.VMEM((ZB,), jnp.float32),
        pltpu.SemaphoreType.DMA,
        pltpu.SemaphoreType.DMA,
    ],
)(_sc_edge_body)


def _tc_pool_body(ap, dp, xr, br, wl, wr, bl, out_ref):
    t = pl.program_id(0)

    @pl.when(t == 0)
    def _():
        out_ref[...] = jnp.full((N_GRAPHS, OUT_F), -jnp.inf, jnp.float32)

    aggr = ap[0, 0] + ap[1, 0]
    deg = dp[0, 0] + dp[1, 0]
    mean = aggr / jnp.maximum(deg, 1.0)
    xv = xr[0]
    b = br[0]

    mcol = mean.reshape(TILE_N, 1)
    xcol = xv.reshape(TILE_N, 1)
    h = mcol * wl[...] + xcol * wr[...]

    bcol = b.reshape(TILE_N, 1)
    g_lo = jnp.minimum(b[0, 0], N_GRAPHS - 1)
    g_hi = jnp.minimum(b[0, TILE_N - 1], N_GRAPHS - 1)
    rows = lax.broadcasted_iota(jnp.int32, (N_GRAPHS, 1), 0)

    def body(g, _):
        mask = bcol == g
        row = jnp.max(jnp.where(mask, h, -jnp.inf), axis=0,
                      keepdims=True)
        cur = out_ref[pl.ds(g, 1), :]
        out_ref[pl.ds(g, 1), :] = jnp.maximum(cur, row)
        return 0
    lax.fori_loop(g_lo, g_hi + 1, body, 0)

    @pl.when(t == pl.num_programs(0) - 1)
    def _():
        out_ref[...] = out_ref[...] + bl[...]


def _tc_pool(ap, dp, xr, br, wl, wr, bl):
    part4 = lambda: pl.BlockSpec((2, 1, 1, TILE_N), lambda t: (0, t, 0, 0))
    node3 = lambda: pl.BlockSpec((1, 1, TILE_N), lambda t: (t, 0, 0))
    const2 = lambda: pl.BlockSpec((1, OUT_F), lambda t: (0, 0))
    return pl.pallas_call(
        _tc_pool_body,
        grid=(N_TILES,),
        in_specs=[part4(), part4(), node3(), node3(),
                  const2(), const2(), const2()],
        out_specs=pl.BlockSpec((N_GRAPHS, OUT_F), lambda t: (0, 0)),
        out_shape=jax.ShapeDtypeStruct((N_GRAPHS, OUT_F), jnp.float32),
    )(ap, dp, xr, br, wl, wr, bl)


def kernel(x, edge_index, batch, W_l, b_l, W_r):
    x_flat = x.reshape(N_NODES)
    x_p = jnp.pad(x_flat, (0, N_PAD - N_NODES))

    dz = jnp.zeros((LANES,), jnp.float32)
    aggr_parts, deg_parts = _sc_edge_kernel(x_p, edge_index, dz)

    shape4 = (2, N_TILES, 1, TILE_N)
    ap = aggr_parts.reshape(shape4)
    dp = deg_parts.reshape(shape4)
    shape3 = (N_TILES, 1, TILE_N)
    xr = x_p.reshape(shape3)
    br = jnp.pad(batch, (0, N_PAD - N_NODES),
                 constant_values=N_GRAPHS).reshape(shape3)

    out = _tc_pool(ap, dp, xr, br,
                   W_l.reshape(1, OUT_F), W_r.reshape(1, OUT_F),
                   b_l.reshape(1, OUT_F))
    return out
